# baseline (speedup 1.0000x reference)
import functools

import jax
import jax.numpy as jnp
from jax import lax
from jax.experimental import pallas as pl
from jax.experimental.pallas import tpu as pltpu
from jax.experimental.pallas import tpu_sc as plsc

NUM_TIME_STEPS = 1000
BATCH = 16384
_NC = 2
_NS = 16
_NW = _NC * _NS
_PER = BATCH // _NW
_L = 16


def _sc_gather_kernel(t_hbm, beta_hbm, alpha_hbm, beta_out, alpha_out,
                      idx_v, obeta_v, oalpha_v, sem_b, sem_a):
    wid = lax.axis_index("s") * _NC + lax.axis_index("c")
    base = wid * _PER

    pltpu.sync_copy(t_hbm.at[pl.ds(base, _PER)], idx_v)

    cb = pltpu.async_copy(beta_hbm.at[idx_v], obeta_v, sem_b)
    ca = pltpu.async_copy(alpha_hbm.at[idx_v], oalpha_v, sem_a)
    cb.wait()
    ca.wait()

    ob = pltpu.async_copy(obeta_v, beta_out.at[pl.ds(base, _PER)], sem_b)
    oa = pltpu.async_copy(oalpha_v, alpha_out.at[pl.ds(base, _PER)], sem_a)
    ob.wait()
    oa.wait()


@jax.jit
def kernel(t, beta, alpha):
    mesh = plsc.VectorSubcoreMesh(core_axis_name="c", subcore_axis_name="s")
    out_t = (
        jax.ShapeDtypeStruct((BATCH,), jnp.float32),
        jax.ShapeDtypeStruct((BATCH,), jnp.float32),
    )
    run = functools.partial(
        pl.kernel,
        mesh=mesh,
        out_type=out_t,
        scratch_types=[
            pltpu.VMEM((_PER,), jnp.int32),
            pltpu.VMEM((_PER,), jnp.float32),
            pltpu.VMEM((_PER,), jnp.float32),
            pltpu.SemaphoreType.DMA,
            pltpu.SemaphoreType.DMA,
        ],
        compiler_params=pltpu.CompilerParams(needs_layout_passes=False),
    )(_sc_gather_kernel)
    return run(t.astype(jnp.int32), beta, alpha)


# speedup vs baseline: 1.4420x; 1.4420x over previous
import functools

import jax
import jax.numpy as jnp
from jax import lax
from jax.experimental import pallas as pl
from jax.experimental.pallas import tpu as pltpu
from jax.experimental.pallas import tpu_sc as plsc

NUM_TIME_STEPS = 1000
BATCH = 16384
_NS = 16
_PER = BATCH // _NS
_L = 16


def _sc_gather_kernel(t_hbm, beta_hbm, alpha_hbm, beta_out, alpha_out,
                      idx_v, beta_v, alpha_v, obeta_v, oalpha_v, sem):
    base = lax.axis_index("s") * _PER

    in_copies = [
        pltpu.async_copy(t_hbm.at[pl.ds(base, _PER)], idx_v, sem),
        pltpu.async_copy(beta_hbm, beta_v, sem),
        pltpu.async_copy(alpha_hbm, alpha_v, sem),
    ]
    for cp in in_copies:
        cp.wait()

    for i in range(_PER // _L):
        off = i * _L
        idx = idx_v[pl.ds(off, _L)]
        obeta_v[pl.ds(off, _L)] = plsc.load_gather(beta_v, [idx])
    ob = pltpu.async_copy(obeta_v, beta_out.at[pl.ds(base, _PER)], sem)

    for i in range(_PER // _L):
        off = i * _L
        idx = idx_v[pl.ds(off, _L)]
        oalpha_v[pl.ds(off, _L)] = plsc.load_gather(alpha_v, [idx])
    oa = pltpu.async_copy(oalpha_v, alpha_out.at[pl.ds(base, _PER)], sem)

    ob.wait()
    oa.wait()


@jax.jit
def kernel(t, beta, alpha):
    mesh = plsc.VectorSubcoreMesh(core_axis_name="c", subcore_axis_name="s",
                                  num_cores=1)
    out_t = (
        jax.ShapeDtypeStruct((BATCH,), jnp.float32),
        jax.ShapeDtypeStruct((BATCH,), jnp.float32),
    )
    run = functools.partial(
        pl.kernel,
        mesh=mesh,
        out_type=out_t,
        scratch_types=[
            pltpu.VMEM((_PER,), jnp.int32),
            pltpu.VMEM((NUM_TIME_STEPS,), jnp.float32),
            pltpu.VMEM((NUM_TIME_STEPS,), jnp.float32),
            pltpu.VMEM((_PER,), jnp.float32),
            pltpu.VMEM((_PER,), jnp.float32),
            pltpu.SemaphoreType.DMA,
        ],
        compiler_params=pltpu.CompilerParams(needs_layout_passes=False),
    )(_sc_gather_kernel)
    return run(t.astype(jnp.int32), beta, alpha)
